# baseline (speedup 1.0000x reference)
import jax
import jax.numpy as jnp
from jax.experimental import pallas as pl
from jax.experimental.pallas import tpu as pltpu

_LRELU_SLOPE = 0.1
_BN_EPS = 1e-5
_VMEM_LIMIT = 56 * 1024 * 1024


def _up_mm_kernel(x_ref, w_ref, b_ref, o_ref):
    x = x_ref[0].astype(jnp.bfloat16)
    acc = jnp.dot(w_ref[...], x, preferred_element_type=jnp.float32)
    o_ref[0] = (acc + b_ref[...]).astype(o_ref.dtype)


def _conv1_stats_kernel(up_ref, fd_ref, w_ref, b_ref, y_ref, s_ref, q_ref):
    _, h, w, c = up_ref.shape
    up_p = jnp.pad(up_ref[0], ((1, 1), (1, 1), (0, 0)))
    fd_p = jnp.pad(fd_ref[0], ((1, 1), (1, 1), (0, 0)))
    m = h * w
    cols = []
    for i in range(3):
        for j in range(3):
            cols.append(up_p[i:i + h, j:j + w, :].reshape(m, c))
            cols.append(fd_p[i:i + h, j:j + w, :].reshape(m, c))
    patches = jnp.concatenate(cols, axis=-1)
    acc = jnp.dot(patches, w_ref[...], preferred_element_type=jnp.float32)
    acc = acc + b_ref[...]
    y_ref[0] = acc.reshape(h, w, -1).astype(y_ref.dtype)
    s_ref[0] = jnp.sum(acc, axis=0, keepdims=True)
    q_ref[0] = jnp.sum(acc * acc, axis=0, keepdims=True)


def _bn_conv2_stats_kernel(y1_ref, sc_ref, sh_ref, w_ref, b_ref,
                           y_ref, s_ref, q_ref):
    _, h, w, c = y1_ref.shape
    z = (y1_ref[0].astype(jnp.float32) * sc_ref[...].reshape(1, 1, c)
         + sh_ref[...].reshape(1, 1, c))
    a = jnp.where(z >= 0, z, _LRELU_SLOPE * z).astype(jnp.bfloat16)
    a_p = jnp.pad(a, ((1, 1), (1, 1), (0, 0)))
    m = h * w
    cols = [a_p[i:i + h, j:j + w, :].reshape(m, c)
            for i in range(3) for j in range(3)]
    patches = jnp.concatenate(cols, axis=-1)
    acc = jnp.dot(patches, w_ref[...], preferred_element_type=jnp.float32)
    acc = acc + b_ref[...]
    y_ref[0] = acc.reshape(h, w, -1).astype(y_ref.dtype)
    s_ref[0] = jnp.sum(acc, axis=0, keepdims=True)
    q_ref[0] = jnp.sum(acc * acc, axis=0, keepdims=True)


def _bn_lrelu_out_kernel(y_ref, sc_ref, sh_ref, o_ref):
    z = y_ref[0].astype(jnp.float32) * sc_ref[...] + sh_ref[...]
    o_ref[0] = jnp.where(z >= 0, z, _LRELU_SLOPE * z)


def _scale_shift(s_part, q_part, gamma, beta, count):
    ssum = jnp.sum(s_part[:, 0, :], axis=0)
    qsum = jnp.sum(q_part[:, 0, :], axis=0)
    mean = ssum / count
    var = jnp.maximum(qsum / count - mean * mean, 0.0)
    scale = gamma / jnp.sqrt(var + _BN_EPS)
    shift = beta - mean * scale
    return scale.astype(jnp.float32), shift.astype(jnp.float32)


def _params(sem):
    return pltpu.CompilerParams(dimension_semantics=(sem,),
                                vmem_limit_bytes=_VMEM_LIMIT)


def kernel(from_down, from_up, up_w, up_b, w1, b1, gamma1, beta1,
           w2, b2, gamma2, beta2):
    n, cin, h, w = from_up.shape
    cout = up_w.shape[-1]
    hh, ww = 2 * h, 2 * w
    bf = jnp.bfloat16

    fu = from_up.reshape(n, cin, h * w)
    wup = jnp.transpose(up_w, (3, 0, 1, 2))
    wup = jnp.transpose(wup, (1, 2, 0, 3)).reshape(4 * cout, cin)
    bup = jnp.tile(up_b.reshape(1, cout), (4, 1)).reshape(4 * cout, 1)

    u = pl.pallas_call(
        _up_mm_kernel,
        out_shape=jax.ShapeDtypeStruct((n, 4 * cout, h * w), bf),
        grid=(n,),
        in_specs=[
            pl.BlockSpec((1, cin, h * w), lambda i: (i, 0, 0)),
            pl.BlockSpec((4 * cout, cin), lambda i: (0, 0)),
            pl.BlockSpec((4 * cout, 1), lambda i: (0, 0)),
        ],
        out_specs=pl.BlockSpec((1, 4 * cout, h * w), lambda i: (i, 0, 0)),
        compiler_params=_params("parallel"),
    )(fu, wup.astype(bf), bup.astype(jnp.float32))

    up = u.reshape(n, 2, 2, cout, h, w).transpose(0, 4, 1, 5, 2, 3)
    up = up.reshape(n, hh, ww, cout)
    fd = jnp.transpose(from_down, (0, 2, 3, 1)).astype(bf)

    w1r = w1.reshape(9 * 2 * cout, cout).astype(bf)
    b1r = b1.reshape(1, cout).astype(jnp.float32)
    y1, s1, q1 = pl.pallas_call(
        _conv1_stats_kernel,
        out_shape=(
            jax.ShapeDtypeStruct((n, hh, ww, cout), bf),
            jax.ShapeDtypeStruct((n, 1, cout), jnp.float32),
            jax.ShapeDtypeStruct((n, 1, cout), jnp.float32),
        ),
        grid=(n,),
        in_specs=[
            pl.BlockSpec((1, hh, ww, cout), lambda i: (i, 0, 0, 0)),
            pl.BlockSpec((1, hh, ww, cout), lambda i: (i, 0, 0, 0)),
            pl.BlockSpec((9 * 2 * cout, cout), lambda i: (0, 0)),
            pl.BlockSpec((1, cout), lambda i: (0, 0)),
        ],
        out_specs=[
            pl.BlockSpec((1, hh, ww, cout), lambda i: (i, 0, 0, 0)),
            pl.BlockSpec((1, 1, cout), lambda i: (i, 0, 0)),
            pl.BlockSpec((1, 1, cout), lambda i: (i, 0, 0)),
        ],
        compiler_params=_params("parallel"),
    )(up, fd, w1r, b1r)

    count = jnp.float32(n * hh * ww)
    sc1, sh1 = _scale_shift(s1, q1, gamma1, beta1, count)

    w2r = w2.reshape(9 * cout, cout).astype(bf)
    b2r = b2.reshape(1, cout).astype(jnp.float32)
    y2, s2, q2 = pl.pallas_call(
        _bn_conv2_stats_kernel,
        out_shape=(
            jax.ShapeDtypeStruct((n, hh, ww, cout), bf),
            jax.ShapeDtypeStruct((n, 1, cout), jnp.float32),
            jax.ShapeDtypeStruct((n, 1, cout), jnp.float32),
        ),
        grid=(n,),
        in_specs=[
            pl.BlockSpec((1, hh, ww, cout), lambda i: (i, 0, 0, 0)),
            pl.BlockSpec((1, cout), lambda i: (0, 0)),
            pl.BlockSpec((1, cout), lambda i: (0, 0)),
            pl.BlockSpec((9 * cout, cout), lambda i: (0, 0)),
            pl.BlockSpec((1, cout), lambda i: (0, 0)),
        ],
        out_specs=[
            pl.BlockSpec((1, hh, ww, cout), lambda i: (i, 0, 0, 0)),
            pl.BlockSpec((1, 1, cout), lambda i: (i, 0, 0)),
            pl.BlockSpec((1, 1, cout), lambda i: (i, 0, 0)),
        ],
        compiler_params=_params("parallel"),
    )(y1, sc1.reshape(1, cout), sh1.reshape(1, cout), w2r, b2r)

    sc2, sh2 = _scale_shift(s2, q2, gamma2, beta2, count)

    y2t = jnp.transpose(y2, (0, 3, 1, 2))
    sc2b = jnp.broadcast_to(sc2.reshape(cout, 1, 1), (cout, 1, ww))
    sh2b = jnp.broadcast_to(sh2.reshape(cout, 1, 1), (cout, 1, ww))
    out = pl.pallas_call(
        _bn_lrelu_out_kernel,
        out_shape=jax.ShapeDtypeStruct((n, cout, hh, ww), jnp.float32),
        grid=(n,),
        in_specs=[
            pl.BlockSpec((1, cout, hh, ww), lambda i: (i, 0, 0, 0)),
            pl.BlockSpec((cout, 1, ww), lambda i: (0, 0, 0)),
            pl.BlockSpec((cout, 1, ww), lambda i: (0, 0, 0)),
        ],
        out_specs=pl.BlockSpec((1, cout, hh, ww), lambda i: (i, 0, 0, 0)),
        compiler_params=_params("parallel"),
    )(y2t, sc2b, sh2b)

    return out


# speedup vs baseline: 1.3556x; 1.3556x over previous
import jax
import jax.numpy as jnp
from jax.experimental import pallas as pl
from jax.experimental.pallas import tpu as pltpu

_LRELU_SLOPE = 0.1
_BN_EPS = 1e-5
_VMEM_LIMIT = 56 * 1024 * 1024


def _up_mm_kernel(x_ref, w_ref, b_ref, o_ref):
    x = x_ref[0].astype(jnp.bfloat16)
    acc = jnp.dot(w_ref[...], x, preferred_element_type=jnp.float32)
    o_ref[0] = (acc + b_ref[...]).astype(o_ref.dtype)


def _conv1_stats_kernel(up_ref, fd_ref, w_ref, b_ref, y_ref, s_ref, q_ref):
    _, h, w, c = up_ref.shape
    up_p = jnp.pad(up_ref[0], ((1, 1), (1, 1), (0, 0)))
    fd_p = jnp.pad(fd_ref[0], ((1, 1), (1, 1), (0, 0)))
    m = h * w
    cols = []
    for i in range(3):
        for j in range(3):
            cols.append(up_p[i:i + h, j:j + w, :].reshape(m, c))
            cols.append(fd_p[i:i + h, j:j + w, :].reshape(m, c))
    patches = jnp.concatenate(cols, axis=-1)
    acc = jnp.dot(patches, w_ref[...], preferred_element_type=jnp.float32)
    acc = acc + b_ref[...]
    y_ref[0] = acc.reshape(h, w, -1).astype(y_ref.dtype)
    s_ref[0] = jnp.sum(acc, axis=0, keepdims=True)
    q_ref[0] = jnp.sum(acc * acc, axis=0, keepdims=True)


def _bn_conv2_stats_kernel(y1_ref, sc_ref, sh_ref, w_ref, b_ref,
                           y_ref, s_ref, q_ref):
    _, h, w, c = y1_ref.shape
    z = (y1_ref[0].astype(jnp.float32) * sc_ref[...].reshape(1, 1, c)
         + sh_ref[...].reshape(1, 1, c))
    a = jnp.where(z >= 0, z, _LRELU_SLOPE * z).astype(jnp.bfloat16)
    a_p = jnp.pad(a, ((1, 1), (1, 1), (0, 0)))
    m = h * w
    cols = [a_p[i:i + h, j:j + w, :].reshape(m, c)
            for i in range(3) for j in range(3)]
    patches = jnp.concatenate(cols, axis=-1)
    acc = jnp.dot(patches, w_ref[...], preferred_element_type=jnp.float32)
    acc = acc + b_ref[...]
    y_ref[0] = acc.reshape(h, w, -1).astype(y_ref.dtype)
    s_ref[0] = jnp.sum(acc, axis=0, keepdims=True)
    q_ref[0] = jnp.sum(acc * acc, axis=0, keepdims=True)


def _bn_lrelu_out_kernel(y_ref, sc_ref, sh_ref, o_ref):
    c = y_ref.shape[-1]
    z = (y_ref[0].astype(jnp.float32) * sc_ref[...].reshape(1, 1, c)
         + sh_ref[...].reshape(1, 1, c))
    o_ref[0] = jnp.where(z >= 0, z, _LRELU_SLOPE * z)


def _scale_shift(s_part, q_part, gamma, beta, count):
    ssum = jnp.sum(s_part[:, 0, :], axis=0)
    qsum = jnp.sum(q_part[:, 0, :], axis=0)
    mean = ssum / count
    var = jnp.maximum(qsum / count - mean * mean, 0.0)
    scale = gamma / jnp.sqrt(var + _BN_EPS)
    shift = beta - mean * scale
    return scale.astype(jnp.float32), shift.astype(jnp.float32)


def _params(sem):
    return pltpu.CompilerParams(dimension_semantics=(sem,),
                                vmem_limit_bytes=_VMEM_LIMIT)


def kernel(from_down, from_up, up_w, up_b, w1, b1, gamma1, beta1,
           w2, b2, gamma2, beta2):
    n, cin, h, w = from_up.shape
    cout = up_w.shape[-1]
    hh, ww = 2 * h, 2 * w
    bf = jnp.bfloat16

    fu = from_up.reshape(n, cin, h * w)
    wup = jnp.transpose(up_w, (3, 0, 1, 2))
    wup = jnp.transpose(wup, (1, 2, 0, 3)).reshape(4 * cout, cin)
    bup = jnp.tile(up_b.reshape(1, cout), (4, 1)).reshape(4 * cout, 1)

    u = pl.pallas_call(
        _up_mm_kernel,
        out_shape=jax.ShapeDtypeStruct((n, 4 * cout, h * w), bf),
        grid=(n,),
        in_specs=[
            pl.BlockSpec((1, cin, h * w), lambda i: (i, 0, 0)),
            pl.BlockSpec((4 * cout, cin), lambda i: (0, 0)),
            pl.BlockSpec((4 * cout, 1), lambda i: (0, 0)),
        ],
        out_specs=pl.BlockSpec((1, 4 * cout, h * w), lambda i: (i, 0, 0)),
        compiler_params=_params("parallel"),
    )(fu, wup.astype(bf), bup.astype(jnp.float32))

    up = u.reshape(n, 2, 2, cout, h, w).transpose(0, 4, 1, 5, 2, 3)
    up = up.reshape(n, hh, ww, cout)
    fd = jnp.transpose(from_down, (0, 2, 3, 1)).astype(bf)

    w1r = w1.reshape(9 * 2 * cout, cout).astype(bf)
    b1r = b1.reshape(1, cout).astype(jnp.float32)
    y1, s1, q1 = pl.pallas_call(
        _conv1_stats_kernel,
        out_shape=(
            jax.ShapeDtypeStruct((n, hh, ww, cout), bf),
            jax.ShapeDtypeStruct((n, 1, cout), jnp.float32),
            jax.ShapeDtypeStruct((n, 1, cout), jnp.float32),
        ),
        grid=(n,),
        in_specs=[
            pl.BlockSpec((1, hh, ww, cout), lambda i: (i, 0, 0, 0)),
            pl.BlockSpec((1, hh, ww, cout), lambda i: (i, 0, 0, 0)),
            pl.BlockSpec((9 * 2 * cout, cout), lambda i: (0, 0)),
            pl.BlockSpec((1, cout), lambda i: (0, 0)),
        ],
        out_specs=[
            pl.BlockSpec((1, hh, ww, cout), lambda i: (i, 0, 0, 0)),
            pl.BlockSpec((1, 1, cout), lambda i: (i, 0, 0)),
            pl.BlockSpec((1, 1, cout), lambda i: (i, 0, 0)),
        ],
        compiler_params=_params("parallel"),
    )(up, fd, w1r, b1r)

    count = jnp.float32(n * hh * ww)
    sc1, sh1 = _scale_shift(s1, q1, gamma1, beta1, count)

    w2r = w2.reshape(9 * cout, cout).astype(bf)
    b2r = b2.reshape(1, cout).astype(jnp.float32)
    y2, s2, q2 = pl.pallas_call(
        _bn_conv2_stats_kernel,
        out_shape=(
            jax.ShapeDtypeStruct((n, hh, ww, cout), bf),
            jax.ShapeDtypeStruct((n, 1, cout), jnp.float32),
            jax.ShapeDtypeStruct((n, 1, cout), jnp.float32),
        ),
        grid=(n,),
        in_specs=[
            pl.BlockSpec((1, hh, ww, cout), lambda i: (i, 0, 0, 0)),
            pl.BlockSpec((1, cout), lambda i: (0, 0)),
            pl.BlockSpec((1, cout), lambda i: (0, 0)),
            pl.BlockSpec((9 * cout, cout), lambda i: (0, 0)),
            pl.BlockSpec((1, cout), lambda i: (0, 0)),
        ],
        out_specs=[
            pl.BlockSpec((1, hh, ww, cout), lambda i: (i, 0, 0, 0)),
            pl.BlockSpec((1, 1, cout), lambda i: (i, 0, 0)),
            pl.BlockSpec((1, 1, cout), lambda i: (i, 0, 0)),
        ],
        compiler_params=_params("parallel"),
    )(y1, sc1.reshape(1, cout), sh1.reshape(1, cout), w2r, b2r)

    sc2, sh2 = _scale_shift(s2, q2, gamma2, beta2, count)

    out = pl.pallas_call(
        _bn_lrelu_out_kernel,
        out_shape=jax.ShapeDtypeStruct((n, hh, ww, cout), jnp.float32),
        grid=(n,),
        in_specs=[
            pl.BlockSpec((1, hh, ww, cout), lambda i: (i, 0, 0, 0)),
            pl.BlockSpec((1, cout), lambda i: (0, 0)),
            pl.BlockSpec((1, cout), lambda i: (0, 0)),
        ],
        out_specs=pl.BlockSpec((1, hh, ww, cout), lambda i: (i, 0, 0, 0)),
        compiler_params=_params("parallel"),
    )(y2, sc2.reshape(1, cout), sh2.reshape(1, cout))

    return jnp.transpose(out, (0, 3, 1, 2))


# speedup vs baseline: 1.6695x; 1.2315x over previous
import jax
import jax.numpy as jnp
from jax.experimental import pallas as pl
from jax.experimental.pallas import tpu as pltpu

_LRELU_SLOPE = 0.1
_BN_EPS = 1e-5
_VMEM_LIMIT = 56 * 1024 * 1024


def _mm_bias_kernel(x_ref, w_ref, b_ref, o_ref):
    acc = jnp.dot(x_ref[...], w_ref[...], preferred_element_type=jnp.float32)
    o_ref[...] = (acc + b_ref[...]).astype(o_ref.dtype)


def _conv_rows(planes, w_ref, b_ref, h, w, c_out):
    m = h * w
    cols = []
    for j in range(3):
        for p in planes:
            cols.append(p[:, j:j + w, :].reshape(m, -1))
    patches = jnp.concatenate(cols, axis=-1)
    y3 = jnp.dot(patches, w_ref[...], preferred_element_type=jnp.float32)
    top = jnp.pad(y3[:m - w, 0 * c_out:1 * c_out], ((w, 0), (0, 0)))
    mid = y3[:, 1 * c_out:2 * c_out]
    bot = jnp.pad(y3[w:, 2 * c_out:3 * c_out], ((0, w), (0, 0)))
    return top + mid + bot + b_ref[...]


def _conv1_stats_kernel(up_ref, fd_ref, w_ref, b_ref, y_ref, s_ref, q_ref):
    _, h, w, c = up_ref.shape
    up_p = jnp.pad(up_ref[0], ((0, 0), (1, 1), (0, 0)))
    fd_p = jnp.pad(fd_ref[0], ((0, 0), (1, 1), (0, 0)))
    acc = _conv_rows([up_p, fd_p], w_ref, b_ref, h, w, c)
    y_ref[0] = acc.reshape(h, w, -1).astype(y_ref.dtype)
    s_ref[0] = jnp.sum(acc, axis=0, keepdims=True)
    q_ref[0] = jnp.sum(acc * acc, axis=0, keepdims=True)


def _bn_conv2_stats_kernel(y1_ref, sc_ref, sh_ref, w_ref, b_ref,
                           y_ref, s_ref, q_ref):
    _, h, w, c = y1_ref.shape
    z = (y1_ref[0].astype(jnp.float32) * sc_ref[...].reshape(1, 1, c)
         + sh_ref[...].reshape(1, 1, c))
    a = jnp.where(z >= 0, z, _LRELU_SLOPE * z).astype(jnp.bfloat16)
    a_p = jnp.pad(a, ((0, 0), (1, 1), (0, 0)))
    acc = _conv_rows([a_p], w_ref, b_ref, h, w, c)
    y_ref[0] = acc.reshape(h, w, -1).astype(y_ref.dtype)
    s_ref[0] = jnp.sum(acc, axis=0, keepdims=True)
    q_ref[0] = jnp.sum(acc * acc, axis=0, keepdims=True)


def _bn_lrelu_out_kernel(y_ref, sc_ref, sh_ref, o_ref):
    c = y_ref.shape[-1]
    z = (y_ref[0].astype(jnp.float32) * sc_ref[...].reshape(1, 1, c)
         + sh_ref[...].reshape(1, 1, c))
    o_ref[0] = jnp.where(z >= 0, z, _LRELU_SLOPE * z)


def _scale_shift(s_part, q_part, gamma, beta, count):
    ssum = jnp.sum(s_part[:, 0, :], axis=0)
    qsum = jnp.sum(q_part[:, 0, :], axis=0)
    mean = ssum / count
    var = jnp.maximum(qsum / count - mean * mean, 0.0)
    scale = gamma / jnp.sqrt(var + _BN_EPS)
    shift = beta - mean * scale
    c = gamma.shape[0]
    return scale.reshape(1, c).astype(jnp.float32), \
        shift.reshape(1, c).astype(jnp.float32)


def _conv_weight(w_hwio):
    kh, kw, cin, cout = w_hwio.shape
    return jnp.transpose(w_hwio, (1, 2, 0, 3)).reshape(kw * cin, kh * cout)


def _params(sem):
    return pltpu.CompilerParams(dimension_semantics=(sem,),
                                vmem_limit_bytes=_VMEM_LIMIT)


def kernel(from_down, from_up, up_w, up_b, w1, b1, gamma1, beta1,
           w2, b2, gamma2, beta2):
    n, cin, h, w = from_up.shape
    cout = up_w.shape[-1]
    hh, ww = 2 * h, 2 * w
    bf = jnp.bfloat16

    fu = jnp.transpose(from_up, (0, 2, 3, 1)).reshape(n * h * w, cin)
    wup = jnp.transpose(up_w, (2, 0, 1, 3)).reshape(cin, 4 * cout)
    bup = jnp.tile(up_b, 4).reshape(1, 4 * cout).astype(jnp.float32)

    m1 = n * h * w
    tm = h * w
    u = pl.pallas_call(
        _mm_bias_kernel,
        out_shape=jax.ShapeDtypeStruct((m1, 4 * cout), bf),
        grid=(m1 // tm,),
        in_specs=[
            pl.BlockSpec((tm, cin), lambda i: (i, 0)),
            pl.BlockSpec((cin, 4 * cout), lambda i: (0, 0)),
            pl.BlockSpec((1, 4 * cout), lambda i: (0, 0)),
        ],
        out_specs=pl.BlockSpec((tm, 4 * cout), lambda i: (i, 0)),
        compiler_params=_params("parallel"),
    )(fu.astype(bf), wup.astype(bf), bup)

    up = u.reshape(n, h, w, 2, 2, cout).transpose(0, 1, 3, 2, 4, 5)
    up = up.reshape(n, hh, ww, cout)
    fd = jnp.transpose(from_down, (0, 2, 3, 1)).astype(bf)

    w1r = _conv_weight(w1).astype(bf)
    b1r = b1.reshape(1, cout).astype(jnp.float32)
    y1, s1, q1 = pl.pallas_call(
        _conv1_stats_kernel,
        out_shape=(
            jax.ShapeDtypeStruct((n, hh, ww, cout), bf),
            jax.ShapeDtypeStruct((n, 1, cout), jnp.float32),
            jax.ShapeDtypeStruct((n, 1, cout), jnp.float32),
        ),
        grid=(n,),
        in_specs=[
            pl.BlockSpec((1, hh, ww, cout), lambda i: (i, 0, 0, 0)),
            pl.BlockSpec((1, hh, ww, cout), lambda i: (i, 0, 0, 0)),
            pl.BlockSpec((3 * 2 * cout, 3 * cout), lambda i: (0, 0)),
            pl.BlockSpec((1, cout), lambda i: (0, 0)),
        ],
        out_specs=[
            pl.BlockSpec((1, hh, ww, cout), lambda i: (i, 0, 0, 0)),
            pl.BlockSpec((1, 1, cout), lambda i: (i, 0, 0)),
            pl.BlockSpec((1, 1, cout), lambda i: (i, 0, 0)),
        ],
        compiler_params=_params("parallel"),
    )(up, fd, w1r, b1r)

    count = jnp.float32(n * hh * ww)
    sc1, sh1 = _scale_shift(s1, q1, gamma1, beta1, count)

    w2r = _conv_weight(w2).astype(bf)
    b2r = b2.reshape(1, cout).astype(jnp.float32)
    y2, s2, q2 = pl.pallas_call(
        _bn_conv2_stats_kernel,
        out_shape=(
            jax.ShapeDtypeStruct((n, hh, ww, cout), bf),
            jax.ShapeDtypeStruct((n, 1, cout), jnp.float32),
            jax.ShapeDtypeStruct((n, 1, cout), jnp.float32),
        ),
        grid=(n,),
        in_specs=[
            pl.BlockSpec((1, hh, ww, cout), lambda i: (i, 0, 0, 0)),
            pl.BlockSpec((1, cout), lambda i: (0, 0)),
            pl.BlockSpec((1, cout), lambda i: (0, 0)),
            pl.BlockSpec((3 * cout, 3 * cout), lambda i: (0, 0)),
            pl.BlockSpec((1, cout), lambda i: (0, 0)),
        ],
        out_specs=[
            pl.BlockSpec((1, hh, ww, cout), lambda i: (i, 0, 0, 0)),
            pl.BlockSpec((1, 1, cout), lambda i: (i, 0, 0)),
            pl.BlockSpec((1, 1, cout), lambda i: (i, 0, 0)),
        ],
        compiler_params=_params("parallel"),
    )(y1, sc1, sh1, w2r, b2r)

    sc2, sh2 = _scale_shift(s2, q2, gamma2, beta2, count)

    out = pl.pallas_call(
        _bn_lrelu_out_kernel,
        out_shape=jax.ShapeDtypeStruct((n, hh, ww, cout), jnp.float32),
        grid=(n,),
        in_specs=[
            pl.BlockSpec((1, hh, ww, cout), lambda i: (i, 0, 0, 0)),
            pl.BlockSpec((1, cout), lambda i: (0, 0)),
            pl.BlockSpec((1, cout), lambda i: (0, 0)),
        ],
        out_specs=pl.BlockSpec((1, hh, ww, cout), lambda i: (i, 0, 0, 0)),
        compiler_params=_params("parallel"),
    )(y2, sc2, sh2)

    return jnp.transpose(out, (0, 3, 1, 2))


# speedup vs baseline: 2.1545x; 1.2905x over previous
import jax
import jax.numpy as jnp
from jax.experimental import pallas as pl
from jax.experimental.pallas import tpu as pltpu

_LRELU_SLOPE = 0.1
_BN_EPS = 1e-5
_VMEM_LIMIT = 56 * 1024 * 1024


def _up_shuffle_kernel(x_ref, w_ref, b_ref, o_ref):
    _, _, hw = x_ref.shape
    _, c4 = w_ref.shape
    cout = c4 // 4
    _, hh, ww, _ = o_ref.shape
    h, w = hh // 2, ww // 2
    x = x_ref[0].astype(jnp.bfloat16)
    acc = jax.lax.dot_general(
        x, w_ref[...], (((0,), (0,)), ((), ())),
        preferred_element_type=jnp.float32)
    acc = acc + b_ref[...]
    for dy in range(2):
        for dx in range(2):
            t = 2 * dy + dx
            tap = acc[:, t * cout:(t + 1) * cout]
            o_ref[0, dy::2, dx::2, :] = tap.reshape(h, w, cout)


def _conv_rows(planes, w_ref, b_ref, h, w, c_out):
    m = h * w
    cols = []
    for j in range(3):
        for p in planes:
            cols.append(p[:, j:j + w, :].reshape(m, -1))
    patches = jnp.concatenate(cols, axis=-1)
    y3 = jnp.dot(patches, w_ref[...], preferred_element_type=jnp.float32)
    top = jnp.pad(y3[:m - w, 0 * c_out:1 * c_out], ((w, 0), (0, 0)))
    mid = y3[:, 1 * c_out:2 * c_out]
    bot = jnp.pad(y3[w:, 2 * c_out:3 * c_out], ((0, w), (0, 0)))
    return top + mid + bot + b_ref[...]


def _conv1_stats_kernel(up_ref, fd_ref, w_ref, b_ref, y_ref, s_ref, q_ref):
    _, h, w, c = up_ref.shape
    up_p = jnp.pad(up_ref[0].astype(jnp.bfloat16), ((0, 0), (1, 1), (0, 0)))
    fd_p = jnp.pad(fd_ref[0], ((0, 0), (1, 1), (0, 0)))
    acc = _conv_rows([up_p, fd_p], w_ref, b_ref, h, w, c)
    y_ref[0] = acc.reshape(h, w, -1).astype(y_ref.dtype)
    s_ref[0] = jnp.sum(acc, axis=0, keepdims=True)
    q_ref[0] = jnp.sum(acc * acc, axis=0, keepdims=True)


def _bn_conv2_stats_kernel(y1_ref, sc_ref, sh_ref, w_ref, b_ref,
                           y_ref, s_ref, q_ref):
    _, h, w, c = y1_ref.shape
    z = (y1_ref[0].astype(jnp.float32) * sc_ref[...].reshape(1, 1, c)
         + sh_ref[...].reshape(1, 1, c))
    a = jnp.where(z >= 0, z, _LRELU_SLOPE * z).astype(jnp.bfloat16)
    a_p = jnp.pad(a, ((0, 0), (1, 1), (0, 0)))
    acc = _conv_rows([a_p], w_ref, b_ref, h, w, c)
    y_ref[0] = acc.reshape(h, w, -1).astype(y_ref.dtype)
    s_ref[0] = jnp.sum(acc, axis=0, keepdims=True)
    q_ref[0] = jnp.sum(acc * acc, axis=0, keepdims=True)


def _bn_lrelu_out_kernel(y_ref, sc_ref, sh_ref, o_ref):
    c = y_ref.shape[-1]
    z = (y_ref[0].astype(jnp.float32) * sc_ref[...].reshape(1, 1, c)
         + sh_ref[...].reshape(1, 1, c))
    o_ref[0] = jnp.where(z >= 0, z, _LRELU_SLOPE * z)


def _scale_shift(s_part, q_part, gamma, beta, count):
    ssum = jnp.sum(s_part[:, 0, :], axis=0)
    qsum = jnp.sum(q_part[:, 0, :], axis=0)
    mean = ssum / count
    var = jnp.maximum(qsum / count - mean * mean, 0.0)
    scale = gamma / jnp.sqrt(var + _BN_EPS)
    shift = beta - mean * scale
    c = gamma.shape[0]
    return scale.reshape(1, c).astype(jnp.float32), \
        shift.reshape(1, c).astype(jnp.float32)


def _conv_weight(w_hwio):
    kh, kw, cin, cout = w_hwio.shape
    return jnp.transpose(w_hwio, (1, 2, 0, 3)).reshape(kw * cin, kh * cout)


def _params(sem):
    return pltpu.CompilerParams(dimension_semantics=(sem,),
                                vmem_limit_bytes=_VMEM_LIMIT)


def kernel(from_down, from_up, up_w, up_b, w1, b1, gamma1, beta1,
           w2, b2, gamma2, beta2):
    n, cin, h, w = from_up.shape
    cout = up_w.shape[-1]
    hh, ww = 2 * h, 2 * w
    bf = jnp.bfloat16

    fu = from_up.reshape(n, cin, h * w)
    wup = jnp.transpose(up_w, (2, 0, 1, 3)).reshape(cin, 4 * cout)
    bup = jnp.tile(up_b, 4).reshape(1, 4 * cout).astype(jnp.float32)

    up = pl.pallas_call(
        _up_shuffle_kernel,
        out_shape=jax.ShapeDtypeStruct((n, hh, ww, cout), jnp.float32),
        grid=(n,),
        in_specs=[
            pl.BlockSpec((1, cin, h * w), lambda i: (i, 0, 0)),
            pl.BlockSpec((cin, 4 * cout), lambda i: (0, 0)),
            pl.BlockSpec((1, 4 * cout), lambda i: (0, 0)),
        ],
        out_specs=pl.BlockSpec((1, hh, ww, cout), lambda i: (i, 0, 0, 0)),
        compiler_params=_params("parallel"),
    )(fu, wup.astype(bf), bup)

    fd = jnp.transpose(from_down, (0, 2, 3, 1)).astype(bf)

    w1r = _conv_weight(w1).astype(bf)
    b1r = b1.reshape(1, cout).astype(jnp.float32)
    y1, s1, q1 = pl.pallas_call(
        _conv1_stats_kernel,
        out_shape=(
            jax.ShapeDtypeStruct((n, hh, ww, cout), bf),
            jax.ShapeDtypeStruct((n, 1, cout), jnp.float32),
            jax.ShapeDtypeStruct((n, 1, cout), jnp.float32),
        ),
        grid=(n,),
        in_specs=[
            pl.BlockSpec((1, hh, ww, cout), lambda i: (i, 0, 0, 0)),
            pl.BlockSpec((1, hh, ww, cout), lambda i: (i, 0, 0, 0)),
            pl.BlockSpec((3 * 2 * cout, 3 * cout), lambda i: (0, 0)),
            pl.BlockSpec((1, cout), lambda i: (0, 0)),
        ],
        out_specs=[
            pl.BlockSpec((1, hh, ww, cout), lambda i: (i, 0, 0, 0)),
            pl.BlockSpec((1, 1, cout), lambda i: (i, 0, 0)),
            pl.BlockSpec((1, 1, cout), lambda i: (i, 0, 0)),
        ],
        compiler_params=_params("parallel"),
    )(up, fd, w1r, b1r)

    count = jnp.float32(n * hh * ww)
    sc1, sh1 = _scale_shift(s1, q1, gamma1, beta1, count)

    w2r = _conv_weight(w2).astype(bf)
    b2r = b2.reshape(1, cout).astype(jnp.float32)
    y2, s2, q2 = pl.pallas_call(
        _bn_conv2_stats_kernel,
        out_shape=(
            jax.ShapeDtypeStruct((n, hh, ww, cout), bf),
            jax.ShapeDtypeStruct((n, 1, cout), jnp.float32),
            jax.ShapeDtypeStruct((n, 1, cout), jnp.float32),
        ),
        grid=(n,),
        in_specs=[
            pl.BlockSpec((1, hh, ww, cout), lambda i: (i, 0, 0, 0)),
            pl.BlockSpec((1, cout), lambda i: (0, 0)),
            pl.BlockSpec((1, cout), lambda i: (0, 0)),
            pl.BlockSpec((3 * cout, 3 * cout), lambda i: (0, 0)),
            pl.BlockSpec((1, cout), lambda i: (0, 0)),
        ],
        out_specs=[
            pl.BlockSpec((1, hh, ww, cout), lambda i: (i, 0, 0, 0)),
            pl.BlockSpec((1, 1, cout), lambda i: (i, 0, 0)),
            pl.BlockSpec((1, 1, cout), lambda i: (i, 0, 0)),
        ],
        compiler_params=_params("parallel"),
    )(y1, sc1, sh1, w2r, b2r)

    sc2, sh2 = _scale_shift(s2, q2, gamma2, beta2, count)

    out = pl.pallas_call(
        _bn_lrelu_out_kernel,
        out_shape=jax.ShapeDtypeStruct((n, hh, ww, cout), jnp.float32),
        grid=(n,),
        in_specs=[
            pl.BlockSpec((1, hh, ww, cout), lambda i: (i, 0, 0, 0)),
            pl.BlockSpec((1, cout), lambda i: (0, 0)),
            pl.BlockSpec((1, cout), lambda i: (0, 0)),
        ],
        out_specs=pl.BlockSpec((1, hh, ww, cout), lambda i: (i, 0, 0, 0)),
        compiler_params=_params("parallel"),
    )(y2, sc2, sh2)

    return jnp.transpose(out, (0, 3, 1, 2))
